# baseline (speedup 1.0000x reference)
import functools

import jax
import jax.numpy as jnp
from jax import lax
from jax.experimental import pallas as pl
from jax.experimental.pallas import tpu as pltpu
from jax.experimental.pallas import tpu_sc as plsc

_N = 10000
_E = 320000
_D = 128
_M = 4
_TEMP = 10.0

_NC = 2
_NS = 16
_NW = _NC * _NS
_CHUNK = 128
_CPW = 80
_EPW = _CHUNK * _CPW
_EPAD = _NW * _EPW
_NPAD = 10240
_RPT = _NPAD // _NS
_DB = _NPAD // _CHUNK
_NB = 2
_SB = 8
_NSB = _CPW // _SB
_SBC0 = 10
_SBC1 = 2 * _NSB - _SBC0
_TOTSB = _NS * (_SBC0 + _SBC1)
_RB = 512


def _sc_agg_body(src_hbm, dst_hbm, x_hbm, zeros_hbm, zflat_hbm,
                 agg_out, deg_out,
                 src_v, dst_v, rows_v, deg_v, agg_s, gsem, ssem, isem):
    cid = lax.axis_index("c")
    sid = lax.axis_index("s")

    nsb = jnp.where(cid == 0, _SBC0, _SBC1)
    sb_base = jnp.where(cid == 0, sid * _SBC0,
                        _NS * _SBC0 + sid * _SBC1)
    cpw = nsb * _SB

    pltpu.sync_copy(zeros_hbm, agg_s.at[pl.ds(sid * _RPT, _RPT)])
    pltpu.sync_copy(zflat_hbm, deg_v)

    def _i_start(s, bb):
        pltpu.async_copy(src_hbm.at[sb_base + s], src_v.at[bb], isem)
        pltpu.async_copy(dst_hbm.at[sb_base + s], dst_v.at[bb], isem)

    def _i_wait(s, bb):
        pltpu.make_async_copy(src_hbm.at[sb_base + s], src_v.at[bb],
                              isem).wait()
        pltpu.make_async_copy(dst_hbm.at[sb_base + s], dst_v.at[bb],
                              isem).wait()

    def _g_start(j):
        sbb = (j >> 3) & 1
        jj = j & 7
        pltpu.async_copy(x_hbm.at[src_v.at[sbb].at[jj]],
                         rows_v.at[j & 1], gsem)

    def _g_wait(j):
        sbb = (j >> 3) & 1
        jj = j & 7
        pltpu.make_async_copy(x_hbm.at[src_v.at[sbb].at[jj]],
                              rows_v.at[j & 1], gsem).wait()

    def _s_start(j):
        sbb = (j >> 3) & 1
        jj = j & 7
        pltpu.async_copy(rows_v.at[j & 1],
                         agg_s.at[dst_v.at[sbb].at[jj]], ssem, add=True)

    def _s_wait(j):
        sbb = (j >> 3) & 1
        jj = j & 7
        pltpu.make_async_copy(rows_v.at[j & 1],
                              agg_s.at[dst_v.at[sbb].at[jj]], ssem).wait()

    ones16 = jnp.full((16,), 1.0, jnp.float32)

    def _hist(j):
        sbb = (j >> 3) & 1
        jj = j & 7

        def _step(k, carry):
            v = dst_v[sbb, jj, pl.ds(k * 16, 16)]
            plsc.addupdate_scatter(deg_v, [v], ones16)
            return carry

        lax.fori_loop(0, _CHUNK // 16, _step, 0)

    _i_start(0, 0)
    _i_wait(0, 0)
    _i_start(1, 1)
    plsc.subcore_barrier()

    _g_start(0)
    _g_start(1)

    _g_wait(0)
    _s_start(0)
    _hist(0)
    _s_wait(0)

    def _main(j, carry):
        jj = j & 7
        _g_wait(j)
        _s_start(j)

        @pl.when((jj == 0) & (j >= _SB) & ((j >> 3) < nsb - 1))
        def _load_next_sb():
            s = (j >> 3) + 1
            _i_start(s, s & 1)

        @pl.when(jj == _SB - 1)
        def _arrive_sb():
            s = (j >> 3) + 1
            _i_wait(s, s & 1)

        _g_start(j + 1)
        _hist(j)
        _s_wait(j)
        return carry

    lax.fori_loop(1, cpw - 1, _main, 0)

    _g_wait(cpw - 1)
    _s_start(cpw - 1)
    _hist(cpw - 1)
    _s_wait(cpw - 1)

    pltpu.sync_copy(deg_v, deg_out.at[cid].at[sid])
    plsc.subcore_barrier()

    pltpu.sync_copy(agg_s.at[pl.ds(sid * _RPT, _RPT)],
                    agg_out.at[cid].at[pl.ds(sid * _RPT, _RPT)])


@functools.cache
def _sc_agg():
  return functools.partial(
    pl.kernel,
    mesh=plsc.VectorSubcoreMesh(core_axis_name="c", subcore_axis_name="s",
                                num_cores=_NC, num_subcores=_NS),
    out_type=[
        jax.ShapeDtypeStruct((_NC, _NPAD, _D), jnp.float32),
        jax.ShapeDtypeStruct((_NC, _NS, _NPAD), jnp.float32),
    ],
    scratch_types=[
        pltpu.VMEM((2, _SB, _CHUNK), jnp.int32),
        pltpu.VMEM((2, _SB, _CHUNK), jnp.int32),
        pltpu.VMEM((_NB, _CHUNK, _D), jnp.float32),
        pltpu.VMEM((_NPAD,), jnp.float32),
        pltpu.VMEM_SHARED((_NPAD, _D), jnp.float32),
        pltpu.SemaphoreType.DMA,
        pltpu.SemaphoreType.DMA,
        pltpu.SemaphoreType.DMA,
    ],
    compiler_params=pltpu.CompilerParams(needs_layout_passes=False),
  )(_sc_agg_body)


def _dense_body(x_ref, agg_ref, deg_ref, wqc_ref, bc_ref, v_ref, o_ref):
    x = x_ref[...]
    logits = jnp.dot(x, wqc_ref[...], preferred_element_type=jnp.float32)
    logits = logits + bc_ref[...]
    mx = jnp.max(logits, axis=-1, keepdims=True)
    ex = jnp.exp(logits - mx)
    choice = ex / jnp.sum(ex, axis=-1, keepdims=True)

    agg = agg_ref[0] + agg_ref[1]
    deg = jnp.sum(deg_ref[...], axis=0)
    w = choice / jnp.maximum(deg, 1.0)
    acc = x
    for m in range(_M):
        tm = jnp.dot(agg, v_ref[m], preferred_element_type=jnp.float32)
        acc = acc + w[:, m:m + 1] * tm
    o_ref[...] = jnp.maximum(acc, 0.0)


def _dense_call(x, agg2, deg3, wqc, bc, V):
    grid = (_N + _RB - 1) // _RB
    return pl.pallas_call(
        _dense_body,
        grid=(grid,),
        in_specs=[
            pl.BlockSpec((_RB, _D), lambda i: (i, 0)),
            pl.BlockSpec((_NC, _RB, _D), lambda i: (0, i, 0)),
            pl.BlockSpec((_NW, _RB, 1), lambda i: (0, i, 0)),
            pl.BlockSpec((_D, _M), lambda i: (0, 0)),
            pl.BlockSpec((1, _M), lambda i: (0, 0)),
            pl.BlockSpec((_M, _D, _D), lambda i: (0, 0, 0)),
        ],
        out_specs=pl.BlockSpec((_RB, _D), lambda i: (i, 0)),
        out_shape=jax.ShapeDtypeStruct((_N, _D), jnp.float32),
    )(x, agg2, deg3, wqc, bc, V)


def kernel(x, edge_index, Wq, bq, Wcode, V):
    src = edge_index[0]
    dst = edge_index[1]
    pad = _EPAD - _E
    src_p = jnp.concatenate(
        [src, jnp.zeros((pad,), jnp.int32)]).reshape(_TOTSB, _SB, _CHUNK)
    dump = _N + jnp.arange(pad, dtype=jnp.int32) % (_NPAD - _N)
    dst_p = jnp.concatenate([dst, dump]).reshape(_TOTSB, _SB, _CHUNK)
    zeros = jnp.zeros((_RPT, _D), jnp.float32)
    zflat = jnp.zeros((_NPAD,), jnp.float32)

    agg2, deg2 = _sc_agg()(src_p, dst_p, x, zeros, zflat)

    wqc = (Wq @ Wcode.T) / _TEMP
    bc = (bq[None, :] @ Wcode.T) / _TEMP

    deg3 = deg2.reshape(_NW, _NPAD, 1)
    return _dense_call(x, agg2, deg3, wqc, bc, V)


# speedup vs baseline: 1.2107x; 1.2107x over previous
import functools

import jax
import jax.numpy as jnp
from jax import lax
from jax.experimental import pallas as pl
from jax.experimental.pallas import tpu as pltpu
from jax.experimental.pallas import tpu_sc as plsc

_N = 10000
_E = 320000
_D = 128
_M = 4
_TEMP = 10.0

_NC = 2
_NS = 16
_NW = _NC * _NS
_CHUNK = 128
_CPW = 79
_EPW = _CHUNK * _CPW
_EPAD = _NW * _EPW
_NPAD = 10240
_RPT = _NPAD // _NS
_RB = 512


def _sc_agg_body(src_hbm, dst_hbm, x_hbm, zeros_hbm, zflat_hbm,
                 agg_out, deg_out,
                 src_v, dst_v, rows_v, deg_v, agg_s, gsem):
    cid = lax.axis_index("c")
    sid = lax.axis_index("s")
    wid = sid * _NC + cid

    pltpu.sync_copy(zeros_hbm, agg_s.at[pl.ds(sid * _RPT, _RPT)])
    pltpu.sync_copy(zflat_hbm, deg_v)
    pltpu.sync_copy(src_hbm.at[wid], src_v)
    pltpu.sync_copy(dst_hbm.at[wid], dst_v)
    plsc.subcore_barrier()

    ones16 = jnp.full((16,), 1.0, jnp.float32)

    def _hist(t, carry):
        j = t // (_CHUNK // 16)
        k = t % (_CHUNK // 16)
        v = dst_v[j, pl.ds(k * 16, 16)]
        plsc.addupdate_scatter(deg_v, [v], ones16)
        return carry

    lax.fori_loop(0, _CPW * (_CHUNK // 16), _hist, 0)

    def _edge_step(j, carry):
        pltpu.async_copy(x_hbm.at[src_v.at[j]], rows_v, gsem).wait()
        pltpu.sync_copy(rows_v, agg_s.at[dst_v.at[j]], add=True)
        return carry

    lax.fori_loop(0, _CPW, _edge_step, 0)

    pltpu.sync_copy(deg_v, deg_out.at[cid].at[sid])
    plsc.subcore_barrier()

    pltpu.sync_copy(agg_s.at[pl.ds(sid * _RPT, _RPT)],
                    agg_out.at[cid].at[pl.ds(sid * _RPT, _RPT)])


@functools.cache
def _sc_agg():
  return functools.partial(
    pl.kernel,
    mesh=plsc.VectorSubcoreMesh(core_axis_name="c", subcore_axis_name="s",
                                num_cores=_NC, num_subcores=_NS),
    out_type=[
        jax.ShapeDtypeStruct((_NC, _NPAD, _D), jnp.float32),
        jax.ShapeDtypeStruct((_NC, _NS, _NPAD), jnp.float32),
    ],
    scratch_types=[
        pltpu.VMEM((_CPW, _CHUNK), jnp.int32),
        pltpu.VMEM((_CPW, _CHUNK), jnp.int32),
        pltpu.VMEM((_CHUNK, _D), jnp.float32),
        pltpu.VMEM((_NPAD,), jnp.float32),
        pltpu.VMEM_SHARED((_NPAD, _D), jnp.float32),
        pltpu.SemaphoreType.DMA,
    ],
    compiler_params=pltpu.CompilerParams(needs_layout_passes=False),
  )(_sc_agg_body)


def _dense_body(x_ref, agg_ref, deg_ref, wqc_ref, bc_ref, v_ref, o_ref):
    x = x_ref[...]
    logits = jnp.dot(x, wqc_ref[...], preferred_element_type=jnp.float32)
    logits = logits + bc_ref[...]
    mx = jnp.max(logits, axis=-1, keepdims=True)
    ex = jnp.exp(logits - mx)
    choice = ex / jnp.sum(ex, axis=-1, keepdims=True)

    agg = agg_ref[0] + agg_ref[1]
    deg = jnp.sum(deg_ref[...], axis=0)
    w = choice / jnp.maximum(deg, 1.0)
    acc = x
    for m in range(_M):
        tm = jnp.dot(agg, v_ref[m], preferred_element_type=jnp.float32)
        acc = acc + w[:, m:m + 1] * tm
    o_ref[...] = jnp.maximum(acc, 0.0)


def _dense_call(x, agg2, deg3, wqc, bc, V):
    grid = (_N + _RB - 1) // _RB
    return pl.pallas_call(
        _dense_body,
        grid=(grid,),
        in_specs=[
            pl.BlockSpec((_RB, _D), lambda i: (i, 0)),
            pl.BlockSpec((_NC, _RB, _D), lambda i: (0, i, 0)),
            pl.BlockSpec((_NW, _RB, 1), lambda i: (0, i, 0)),
            pl.BlockSpec((_D, _M), lambda i: (0, 0)),
            pl.BlockSpec((1, _M), lambda i: (0, 0)),
            pl.BlockSpec((_M, _D, _D), lambda i: (0, 0, 0)),
        ],
        out_specs=pl.BlockSpec((_RB, _D), lambda i: (i, 0)),
        out_shape=jax.ShapeDtypeStruct((_N, _D), jnp.float32),
    )(x, agg2, deg3, wqc, bc, V)


def kernel(x, edge_index, Wq, bq, Wcode, V):
    src = edge_index[0]
    dst = edge_index[1]
    pad = _EPAD - _E
    src_p = jnp.concatenate(
        [src, jnp.zeros((pad,), jnp.int32)]).reshape(_NW, _CPW, _CHUNK)
    dump = _N + jnp.arange(pad, dtype=jnp.int32) % (_NPAD - _N)
    dst_p = jnp.concatenate([dst, dump]).reshape(_NW, _CPW, _CHUNK)
    zeros = jnp.zeros((_RPT, _D), jnp.float32)
    zflat = jnp.zeros((_NPAD,), jnp.float32)

    agg2, deg2 = _sc_agg()(src_p, dst_p, x, zeros, zflat)

    wqc = (Wq @ Wcode.T) / _TEMP
    bc = (bq[None, :] @ Wcode.T) / _TEMP

    deg3 = deg2.reshape(_NW, _NPAD, 1)
    return _dense_call(x, agg2, deg3, wqc, bc, V)
